# baseline (speedup 1.0000x reference)
import functools

import jax
import jax.numpy as jnp
from jax import lax
from jax.experimental import pallas as pl
from jax.experimental.pallas import tpu as pltpu
from jax.experimental.pallas import tpu_sc as plsc

L = 16


def _pos_encoding_table(seq_len, d_model):
    pos = jnp.arange(seq_len, dtype=jnp.float32)[:, None]
    _2i = jnp.arange(0, d_model, 2, dtype=jnp.float32)
    angle = pos / jnp.power(10000.0, _2i / d_model)
    pe = jnp.zeros((seq_len, d_model), dtype=jnp.float32)
    pe = pe.at[:, 0::2].set(jnp.sin(angle))
    pe = pe.at[:, 1::2].set(jnp.cos(angle))
    return pe


def _rsqrt16(v):
    half = v * 0.5
    i = plsc.bitcast(v, jnp.int32)
    i = 0x5F3759DF - (i >> 1)
    r = plsc.bitcast(i, jnp.float32)
    r = r * (1.5 - half * r * r)
    r = r * (1.5 - half * r * r)
    r = r * (1.5 - half * r * r)
    return r


def _make_sc_kernel(B, S, D, V, SB):
    info = plsc.get_sparse_core_info()
    NC, NS = info.num_cores, info.num_subcores
    NW = NC * NS
    assert S % NW == 0
    s_per_w = S // NW
    assert s_per_w % SB == 0
    n_sb = s_per_w // SB
    n_slices = D // L

    mesh = plsc.VectorSubcoreMesh(core_axis_name="c", subcore_axis_name="s")

    @functools.partial(
        pl.kernel,
        mesh=mesh,
        out_type=jax.ShapeDtypeStruct((B, S, D), jnp.float32),
        compiler_params=pltpu.CompilerParams(needs_layout_passes=False),
        scratch_types=[
            pltpu.VMEM((SB,), jnp.int32),
            pltpu.VMEM((SB, D), jnp.float32),
            pltpu.VMEM((SB, D), jnp.float32),
            pltpu.VMEM((D,), jnp.float32),
            pltpu.VMEM((D,), jnp.float32),
            pltpu.SemaphoreType.DMA,
        ],
    )
    def k(table_hbm, ids_hbm, pe_hbm, gamma_hbm, beta_hbm, out_hbm,
          idx_v, pe_v, rows_v, gamma_v, beta_v, sem):
        wid = lax.axis_index("s") * NC + lax.axis_index("c")
        s0 = wid * s_per_w

        pltpu.sync_copy(gamma_hbm, gamma_v)
        pltpu.sync_copy(beta_hbm, beta_v)

        inv_d = jnp.float32(1.0 / D)

        def token_body(t, _):
            acc = jnp.zeros((L,), jnp.float32)
            acc2 = jnp.zeros((L,), jnp.float32)
            for j in range(n_slices):
                sl = pl.ds(j * L, L)
                y = rows_v[t, sl] + pe_v[t, sl]
                rows_v[t, sl] = y
                acc = acc + y
                acc2 = acc2 + y * y
            mean = jnp.sum(acc) * inv_d
            var = jnp.sum(acc2) * inv_d - mean * mean
            meanv = jnp.full((L,), mean, jnp.float32)
            rinv = _rsqrt16(jnp.full((L,), var + jnp.float32(1e-5), jnp.float32))
            for j in range(n_slices):
                sl = pl.ds(j * L, L)
                y = rows_v[t, sl]
                rows_v[t, sl] = (y - meanv) * rinv * gamma_v[sl] + beta_v[sl]
            return 0

        def b_body_at(s_base, b):
            pltpu.sync_copy(ids_hbm.at[b, pl.ds(s_base, SB)], idx_v)
            pltpu.async_copy(table_hbm.at[idx_v], rows_v, sem).wait()
            lax.fori_loop(0, SB, token_body, 0)
            pltpu.sync_copy(rows_v, out_hbm.at[b, pl.ds(s_base, SB)])

        def sb_body(sb, _):
            s_base = s0 + sb * SB
            pltpu.sync_copy(pe_hbm.at[pl.ds(s_base, SB)], pe_v)
            lax.fori_loop(0, B, lambda b, c: (b_body_at(s_base, b), c)[1], 0)
            return 0

        lax.fori_loop(0, n_sb, sb_body, 0)

    return k


def kernel(trg_ids, emb_table, gamma, beta):
    B, S = trg_ids.shape
    V, D = emb_table.shape
    pe = _pos_encoding_table(S, D)
    k = _make_sc_kernel(B, S, D, V, SB=32)
    return k(emb_table, trg_ids.astype(jnp.int32), pe, gamma, beta)


# speedup vs baseline: 1.3479x; 1.3479x over previous
import functools

import jax
import jax.numpy as jnp
from jax import lax
from jax.experimental import pallas as pl
from jax.experimental.pallas import tpu as pltpu
from jax.experimental.pallas import tpu_sc as plsc

L = 16


def _pos_encoding_table(seq_len, d_model):
    pos = jnp.arange(seq_len, dtype=jnp.float32)[:, None]
    _2i = jnp.arange(0, d_model, 2, dtype=jnp.float32)
    angle = pos / jnp.power(10000.0, _2i / d_model)
    pe = jnp.zeros((seq_len, d_model), dtype=jnp.float32)
    pe = pe.at[:, 0::2].set(jnp.sin(angle))
    pe = pe.at[:, 1::2].set(jnp.cos(angle))
    return pe


def _rsqrt16(v):
    half = v * 0.5
    i = plsc.bitcast(v, jnp.int32)
    i = 0x5F3759DF - (i >> 1)
    r = plsc.bitcast(i, jnp.float32)
    r = r * (1.5 - half * r * r)
    r = r * (1.5 - half * r * r)
    r = r * (1.5 - half * r * r)
    return r


def _make_sc_kernel(B, S, D, V, SB):
    info = plsc.get_sparse_core_info()
    NC, NS = info.num_cores, info.num_subcores
    NW = NC * NS
    assert S % NW == 0
    s_per_w = S // NW
    assert s_per_w % SB == 0
    n_sb = s_per_w // SB
    n_slices = D // L

    mesh = plsc.VectorSubcoreMesh(core_axis_name="c", subcore_axis_name="s")

    @functools.partial(
        pl.kernel,
        mesh=mesh,
        out_type=jax.ShapeDtypeStruct((B, S, D), jnp.float32),
        compiler_params=pltpu.CompilerParams(needs_layout_passes=False),
        scratch_types=[
            pltpu.VMEM((SB,), jnp.int32),
            pltpu.VMEM((SB, D), jnp.float32),
            pltpu.VMEM((SB, D), jnp.float32),
            pltpu.VMEM((D,), jnp.float32),
            pltpu.VMEM((D,), jnp.float32),
            pltpu.SemaphoreType.DMA,
        ],
    )
    def k(table_hbm, ids_hbm, pe_hbm, gamma_hbm, beta_hbm, out_hbm,
          idx_v, pe_v, rows_v, gamma_v, beta_v, sem):
        wid = lax.axis_index("s") * NC + lax.axis_index("c")
        s0 = wid * s_per_w

        pltpu.sync_copy(gamma_hbm, gamma_v)
        pltpu.sync_copy(beta_hbm, beta_v)

        inv_d = jnp.float32(1.0 / D)
        lanes = lax.iota(jnp.int32, L)
        zf = jnp.zeros((L,), jnp.float32)

        def chunk_body(g, _):
            sb = g // B
            b = g % B
            s_base = s0 + sb * SB

            @pl.when(b == 0)
            def _():
                pltpu.sync_copy(pe_hbm.at[pl.ds(s_base, SB)], pe_v)

            pltpu.sync_copy(ids_hbm.at[b, pl.ds(s_base, SB)], idx_v)
            pltpu.async_copy(table_hbm.at[idx_v], rows_v, sem).wait()

            def t_body(t, carry):
                sum_vec, ssq_vec = carry
                a = [zf, zf, zf, zf]
                q = [zf, zf, zf, zf]
                for j in range(n_slices):
                    sl = pl.ds(j * L, L)
                    y = rows_v[t, sl] + pe_v[t, sl]
                    rows_v[t, sl] = y
                    a[j % 4] = a[j % 4] + y
                    q[j % 4] = q[j % 4] + y * y
                s = jnp.sum((a[0] + a[1]) + (a[2] + a[3]))
                ss = jnp.sum((q[0] + q[1]) + (q[2] + q[3]))
                lane = lanes == t
                return (jnp.where(lane, s, sum_vec),
                        jnp.where(lane, ss, ssq_vec))

            sum_vec, ssq_vec = lax.fori_loop(0, SB, t_body, (zf, zf))
            mean_vec = sum_vec * inv_d
            var_vec = ssq_vec * inv_d - mean_vec * mean_vec
            rinv_vec = _rsqrt16(var_vec + jnp.float32(1e-5))

            msp = [jnp.full((L,), mean_vec[t], jnp.float32) for t in range(SB)]
            rsp = [jnp.full((L,), rinv_vec[t], jnp.float32) for t in range(SB)]

            def j_body(j, _):
                sl = pl.ds(j * L, L)
                gj = gamma_v[sl]
                bj = beta_v[sl]
                for t in range(SB):
                    y = rows_v[t, sl]
                    rows_v[t, sl] = (y - msp[t]) * rsp[t] * gj + bj
                return 0

            lax.fori_loop(0, n_slices, j_body, 0)

            pltpu.sync_copy(rows_v, out_hbm.at[b, pl.ds(s_base, SB)])
            return 0

        lax.fori_loop(0, n_sb * B, chunk_body, 0)

    return k


def kernel(trg_ids, emb_table, gamma, beta):
    B, S = trg_ids.shape
    V, D = emb_table.shape
    pe = _pos_encoding_table(S, D)
    k = _make_sc_kernel(B, S, D, V, SB=16)
    return k(emb_table, trg_ids.astype(jnp.int32), pe, gamma, beta)
